# Initial kernel scaffold; baseline (speedup 1.0000x reference)
import jax
import jax.numpy as jnp
from jax.experimental import pallas as pl


def kernel(x, edge_index, edge_type, batch, W1, b1, W2, b2, Wrel, Wroot, bconv, Wout, bout):
    raise NotImplementedError("write your pallas kernel here")


# speedup vs baseline: 3.6208x; 3.6208x over previous
import functools

import jax
import jax.numpy as jnp
from jax import lax
from jax.experimental import pallas as pl
from jax.experimental.pallas import tpu as pltpu
from jax.experimental.pallas import tpu_sc as plsc

N = 10000
E = 320000
DIN = 128
H = 64
OUT = 64
R = 41
G = 8
L = 2

NC = 2
NS = 16
NW = NC * NS

HP = 128
NRP = 410112
WSL = NRP // NS
EC = E // NS
EW = E // NW
CHC = 2000
CHM = 128
NFC = EW // CHM
TLM = EW - NFC * CHM
NPAD = 10240
NT = NPAD // NS

_MESH = plsc.VectorSubcoreMesh(
    core_axis_name="c", subcore_axis_name="s", num_cores=NC, num_subcores=NS)


@functools.partial(
    pl.kernel,
    out_type=[
        jax.ShapeDtypeStruct((NC * NRP,), jnp.float32),
        jax.ShapeDtypeStruct((E,), jnp.float32),
        jax.ShapeDtypeStruct((E,), jnp.int32),
    ],
    mesh=_MESH,
    scratch_types=[
        pltpu.MemorySpace.VMEM_SHARED((NRP,), jnp.float32),
        pltpu.MemorySpace.VMEM((CHC,), jnp.int32),
        pltpu.MemorySpace.VMEM((CHC,), jnp.int32),
        pltpu.MemorySpace.VMEM((CHC,), jnp.int32),
        pltpu.MemorySpace.VMEM((CHC,), jnp.int32),
        pltpu.MemorySpace.VMEM((CHC,), jnp.float32),
        pltpu.MemorySpace.VMEM((WSL,), jnp.float32),
        pltpu.SemaphoreType.DMA,
    ],
)
def _sc_precompute(src_hbm, dst_hbm, typ_hbm, wtab_hbm, wedge_hbm, gidx_hbm,
                   cnt_sh, dbuf, tbuf, xbuf, ibuf, fbuf, cbuf, sem):
    c = lax.axis_index("c")
    s = lax.axis_index("s")

    wsl0 = pl.multiple_of(s * WSL, 8)

    def zero16(i, _):
        cbuf[pl.ds(i * 16, 16)] = jnp.zeros((16,), jnp.float32)
        return 0
    lax.fori_loop(0, WSL // 16, zero16, 0)
    pltpu.sync_copy(cbuf, cnt_sh.at[pl.ds(wsl0, WSL)])

    def ones16(i, _):
        fbuf[pl.ds(i * 16, 16)] = jnp.ones((16,), jnp.float32)
        return 0
    lax.fori_loop(0, CHC // 16, ones16, 0)
    plsc.subcore_barrier()

    def count_chunk(j, _):
        base = pl.multiple_of(s * EC + j * CHC, 8)
        pltpu.sync_copy(dst_hbm.at[pl.ds(base, CHC)], dbuf)
        pltpu.sync_copy(typ_hbm.at[pl.ds(base, CHC)], tbuf)

        def seg16(i, _):
            sl = pl.ds(i * 16, 16)
            ibuf[sl] = dbuf[sl] * R + tbuf[sl]
            return 0
        lax.fori_loop(0, CHC // 16, seg16, 0)
        pltpu.sync_copy(fbuf, cnt_sh.at[ibuf], add=True)
        return 0
    lax.fori_loop(0, EC // CHC, count_chunk, 0)
    plsc.subcore_barrier()

    pltpu.sync_copy(cnt_sh.at[pl.ds(wsl0, WSL)], cbuf)

    def recip16(i, _):
        sl = pl.ds(i * 16, 16)
        cbuf[sl] = 1.0 / jnp.maximum(cbuf[sl], 1.0)
        return 0
    lax.fori_loop(0, WSL // 16, recip16, 0)
    wbase = pl.multiple_of(c * NRP + s * WSL, 8)
    pltpu.sync_copy(cbuf, wtab_hbm.at[pl.ds(wbase, WSL)])
    plsc.subcore_barrier()

    wid = s * NC + c

    def edge_chunk(j, _):
        base = pl.multiple_of(wid * EW + j * CHC, 8)
        pltpu.sync_copy(src_hbm.at[pl.ds(base, CHC)], xbuf)
        pltpu.sync_copy(dst_hbm.at[pl.ds(base, CHC)], dbuf)
        pltpu.sync_copy(typ_hbm.at[pl.ds(base, CHC)], tbuf)

        def gi16(i, _):
            sl = pl.ds(i * 16, 16)
            t = tbuf[sl]
            ibuf[sl] = dbuf[sl] * R + t + c * NRP
            xbuf[sl] = t * N + xbuf[sl]
            return 0
        lax.fori_loop(0, CHC // 16, gi16, 0)
        pltpu.sync_copy(xbuf, gidx_hbm.at[pl.ds(base, CHC)])
        pltpu.async_copy(wtab_hbm.at[ibuf], fbuf, sem).wait()
        pltpu.sync_copy(fbuf, wedge_hbm.at[pl.ds(base, CHC)])
        return 0
    lax.fori_loop(0, EW // CHC, edge_chunk, 0)


@functools.partial(
    pl.kernel,
    out_type=jax.ShapeDtypeStruct((NC, NPAD, HP), jnp.float32),
    mesh=_MESH,
    scratch_types=[
        pltpu.MemorySpace.VMEM_SHARED((NPAD, HP), jnp.float32),
        pltpu.MemorySpace.VMEM((CHM, HP), jnp.float32),
        pltpu.MemorySpace.VMEM((CHM,), jnp.int32),
        pltpu.MemorySpace.VMEM((CHM,), jnp.int32),
        pltpu.MemorySpace.VMEM((CHM,), jnp.float32),
        pltpu.MemorySpace.VMEM((TLM,), jnp.int32),
        pltpu.MemorySpace.VMEM((TLM,), jnp.int32),
        pltpu.MemorySpace.VMEM((TLM,), jnp.float32),
        pltpu.SemaphoreType.DMA,
    ],
)
def _sc_aggregate(hw_hbm, gidx_hbm, dst_hbm, wedge_hbm, part_hbm,
                  agg_sh, wide, gbuf, dbuf, wsm, gbt, dbt, wst, sem):
    c = lax.axis_index("c")
    s = lax.axis_index("s")

    def zrow(i, _):
        for k in range(HP // 16):
            wide[i, pl.ds(k * 16, 16)] = jnp.zeros((16,), jnp.float32)
        return 0
    lax.fori_loop(0, CHM, zrow, 0)
    row0 = pl.multiple_of(s * NT, 8)
    for off in range(0, NT, CHM):
        pltpu.sync_copy(wide, agg_sh.at[pl.ds(row0 + off, CHM)])
    plsc.subcore_barrier()

    wid = s * NC + c

    def scale_rows(cn, wref):
        def scale16(i, _):
            wv16 = wref[pl.ds(i * 16, 16)]
            for u in range(16):
                e = i * 16 + u
                wv = wv16[u]
                for k in range(H // 16):
                    sl = pl.ds(k * 16, 16)
                    wide[e, sl] = wide[e, sl] * wv
            return 0
        lax.fori_loop(0, cn // 16, scale16, 0)

    def chunk(j, _):
        base = pl.multiple_of(wid * EW + j * CHM, 8)
        pltpu.sync_copy(gidx_hbm.at[pl.ds(base, CHM)], gbuf)
        pltpu.sync_copy(dst_hbm.at[pl.ds(base, CHM)], dbuf)
        pltpu.sync_copy(wedge_hbm.at[pl.ds(base, CHM)], wsm)
        pltpu.async_copy(hw_hbm.at[gbuf], wide, sem).wait()
        scale_rows(CHM, wsm)
        pltpu.sync_copy(wide, agg_sh.at[dbuf], add=True)
        return 0
    lax.fori_loop(0, NFC, chunk, 0)

    tbase = pl.multiple_of(wid * EW + NFC * CHM, 8)
    pltpu.sync_copy(gidx_hbm.at[pl.ds(tbase, TLM)], gbt)
    pltpu.sync_copy(dst_hbm.at[pl.ds(tbase, TLM)], dbt)
    pltpu.sync_copy(wedge_hbm.at[pl.ds(tbase, TLM)], wst)
    pltpu.async_copy(hw_hbm.at[gbt], wide.at[pl.ds(0, TLM)], sem).wait()
    scale_rows(TLM, wst)
    pltpu.sync_copy(wide.at[pl.ds(0, TLM)], agg_sh.at[dbt], add=True)

    plsc.subcore_barrier()
    pltpu.sync_copy(agg_sh.at[pl.ds(row0, NT)],
                    part_hbm.at[c, pl.ds(row0, NT)])


def _mlp_body(x_ref, w1_ref, b1_ref, w2_ref, b2_ref, o_ref):
    h = jnp.dot(x_ref[...], w1_ref[...], preferred_element_type=jnp.float32)
    h = jnp.maximum(h + b1_ref[...], 0.0)
    o_ref[...] = (jnp.dot(h, w2_ref[...], preferred_element_type=jnp.float32)
                  + b2_ref[...])


def _mlp_call(x, W1, b1, W2, b2):
    return pl.pallas_call(
        _mlp_body,
        out_shape=jax.ShapeDtypeStruct((N, H), jnp.float32),
    )(x, W1, b1, W2, b2)


def _einsum_body(h_ref, w_ref, o_ref):
    hw = jnp.dot(h_ref[...], w_ref[0], preferred_element_type=jnp.float32)
    o_ref[0] = jnp.concatenate(
        [hw, jnp.zeros((N, HP - H), jnp.float32)], axis=1)


def _einsum_call(h, Wrel_l):
    return pl.pallas_call(
        _einsum_body,
        grid=(R,),
        in_specs=[
            pl.BlockSpec((N, H), lambda r: (0, 0)),
            pl.BlockSpec((1, H, H), lambda r: (r, 0, 0)),
        ],
        out_specs=pl.BlockSpec((1, N, HP), lambda r: (r, 0, 0)),
        out_shape=jax.ShapeDtypeStruct((R, N, HP), jnp.float32),
    )(h, Wrel_l)


def _combine_body(part_ref, h_ref, wr_ref, b_ref, o_ref):
    o_ref[...] = (part_ref[0, :, :H] + part_ref[1, :, :H] + b_ref[...]
                  + jnp.dot(h_ref[...], wr_ref[...],
                            preferred_element_type=jnp.float32))


def _combine_call(part, h, Wroot_l, bconv_l):
    return pl.pallas_call(
        _combine_body,
        grid=(1,),
        in_specs=[
            pl.BlockSpec((NC, N, HP), lambda i: (0, 0, 0)),
            pl.BlockSpec((N, H), lambda i: (0, 0)),
            pl.BlockSpec((H, H), lambda i: (0, 0)),
            pl.BlockSpec((1, H), lambda i: (0, 0)),
        ],
        out_specs=pl.BlockSpec((N, H), lambda i: (0, 0)),
        out_shape=jax.ShapeDtypeStruct((N, H), jnp.float32),
    )(part, h, Wroot_l, bconv_l)


def _pool_body(h_ref, bat_ref, wo_ref, bo_ref, o_ref):
    h = h_ref[...]
    bat = bat_ref[...]
    cols = []
    for g in range(G):
        hm = jnp.where(bat == g, h, -jnp.inf)
        cols.append(jnp.max(hm, axis=0, keepdims=True))
    pooled = jnp.concatenate(cols, axis=0)
    o_ref[...] = (jnp.dot(pooled, wo_ref[...],
                          preferred_element_type=jnp.float32) + bo_ref[...])


def _pool_call(h, batch2d, Wout, bout):
    return pl.pallas_call(
        _pool_body,
        out_shape=jax.ShapeDtypeStruct((G, OUT), jnp.float32),
    )(h, batch2d, Wout, bout)


def kernel(x, edge_index, edge_type, batch,
           W1, b1, W2, b2, Wrel, Wroot, bconv, Wout, bout):
    src = edge_index[0]
    dst = edge_index[1]

    h = _mlp_call(x, W1, b1.reshape(1, H), W2, b2.reshape(1, H))
    _, wedge, gidx = _sc_precompute(src, dst, edge_type)

    for l in range(L):
        hw = _einsum_call(h, Wrel[l])
        part = _sc_aggregate(hw.reshape(R * N, HP), gidx, dst, wedge)
        h = _combine_call(part, h, Wroot[l], bconv[l].reshape(1, H))

    return _pool_call(h, batch.reshape(N, 1), Wout, bout.reshape(1, OUT))


# speedup vs baseline: 6.4317x; 1.7763x over previous
import functools

import jax
import jax.numpy as jnp
from jax import lax
from jax.experimental import pallas as pl
from jax.experimental.pallas import tpu as pltpu
from jax.experimental.pallas import tpu_sc as plsc

N = 10000
E = 320000
DIN = 128
H = 64
OUT = 64
R = 41
G = 8
L = 2

NC = 2
NS = 16
NW = NC * NS

HP = 128
NRP = 410112
WSL = NRP // NS
EC = E // NS
EW = E // NW
CHC = 2000
CHM = 128
NFC = EW // CHM
TLM = EW - NFC * CHM
NPAD = 10240
NT = NPAD // NS

_MESH = plsc.VectorSubcoreMesh(
    core_axis_name="c", subcore_axis_name="s", num_cores=NC, num_subcores=NS)


@functools.partial(
    pl.kernel,
    out_type=[
        jax.ShapeDtypeStruct((NC * NRP,), jnp.float32),
        jax.ShapeDtypeStruct((E,), jnp.float32),
        jax.ShapeDtypeStruct((E,), jnp.int32),
    ],
    mesh=_MESH,
    scratch_types=[
        pltpu.MemorySpace.VMEM_SHARED((NRP,), jnp.float32),
        pltpu.MemorySpace.VMEM((CHC,), jnp.int32),
        pltpu.MemorySpace.VMEM((CHC,), jnp.int32),
        pltpu.MemorySpace.VMEM((CHC,), jnp.int32),
        pltpu.MemorySpace.VMEM((CHC,), jnp.int32),
        pltpu.MemorySpace.VMEM((CHC,), jnp.float32),
        pltpu.MemorySpace.VMEM((WSL,), jnp.float32),
        pltpu.SemaphoreType.DMA,
    ],
)
def _sc_precompute(src_hbm, dst_hbm, typ_hbm, wtab_hbm, wedge_hbm, gidx_hbm,
                   cnt_sh, dbuf, tbuf, xbuf, ibuf, fbuf, cbuf, sem):
    c = lax.axis_index("c")
    s = lax.axis_index("s")

    wsl0 = pl.multiple_of(s * WSL, 8)

    def zero16(i, _):
        cbuf[pl.ds(i * 16, 16)] = jnp.zeros((16,), jnp.float32)
        return 0
    lax.fori_loop(0, WSL // 16, zero16, 0)
    pltpu.sync_copy(cbuf, cnt_sh.at[pl.ds(wsl0, WSL)])

    def ones16(i, _):
        fbuf[pl.ds(i * 16, 16)] = jnp.ones((16,), jnp.float32)
        return 0
    lax.fori_loop(0, CHC // 16, ones16, 0)
    plsc.subcore_barrier()

    def count_chunk(j, _):
        base = pl.multiple_of(s * EC + j * CHC, 8)
        pltpu.sync_copy(dst_hbm.at[pl.ds(base, CHC)], dbuf)
        pltpu.sync_copy(typ_hbm.at[pl.ds(base, CHC)], tbuf)

        def seg16(i, _):
            sl = pl.ds(i * 16, 16)
            ibuf[sl] = dbuf[sl] * R + tbuf[sl]
            return 0
        lax.fori_loop(0, CHC // 16, seg16, 0)
        pltpu.sync_copy(fbuf, cnt_sh.at[ibuf], add=True)
        return 0
    lax.fori_loop(0, EC // CHC, count_chunk, 0)
    plsc.subcore_barrier()

    pltpu.sync_copy(cnt_sh.at[pl.ds(wsl0, WSL)], cbuf)

    def recip16(i, _):
        sl = pl.ds(i * 16, 16)
        cbuf[sl] = 1.0 / jnp.maximum(cbuf[sl], 1.0)
        return 0
    lax.fori_loop(0, WSL // 16, recip16, 0)
    wbase = pl.multiple_of(c * NRP + s * WSL, 8)
    pltpu.sync_copy(cbuf, wtab_hbm.at[pl.ds(wbase, WSL)])
    plsc.subcore_barrier()

    wid = s * NC + c

    def edge_chunk(j, _):
        base = pl.multiple_of(wid * EW + j * CHC, 8)
        pltpu.sync_copy(src_hbm.at[pl.ds(base, CHC)], xbuf)
        pltpu.sync_copy(dst_hbm.at[pl.ds(base, CHC)], dbuf)
        pltpu.sync_copy(typ_hbm.at[pl.ds(base, CHC)], tbuf)

        def gi16(i, _):
            sl = pl.ds(i * 16, 16)
            t = tbuf[sl]
            ibuf[sl] = dbuf[sl] * R + t + c * NRP
            xbuf[sl] = t * N + xbuf[sl]
            return 0
        lax.fori_loop(0, CHC // 16, gi16, 0)
        pltpu.sync_copy(xbuf, gidx_hbm.at[pl.ds(base, CHC)])
        pltpu.async_copy(wtab_hbm.at[ibuf], fbuf, sem).wait()
        pltpu.sync_copy(fbuf, wedge_hbm.at[pl.ds(base, CHC)])
        return 0
    lax.fori_loop(0, EW // CHC, edge_chunk, 0)


@functools.partial(
    pl.kernel,
    out_type=jax.ShapeDtypeStruct((NC, NPAD, HP), jnp.float32),
    mesh=_MESH,
    scratch_types=[
        pltpu.MemorySpace.VMEM_SHARED((NPAD, HP), jnp.float32),
        pltpu.MemorySpace.VMEM((2, CHM, HP), jnp.float32),
        pltpu.MemorySpace.VMEM((2, CHM), jnp.int32),
        pltpu.MemorySpace.VMEM((2, CHM), jnp.int32),
        pltpu.MemorySpace.VMEM((2, CHM), jnp.int32),
        pltpu.MemorySpace.VMEM((2, CHM), jnp.float32),
        pltpu.MemorySpace.VMEM((TLM,), jnp.int32),
        pltpu.MemorySpace.VMEM((TLM,), jnp.int32),
        pltpu.MemorySpace.VMEM((TLM,), jnp.float32),
        pltpu.SemaphoreType.DMA,
        pltpu.SemaphoreType.DMA,
        pltpu.SemaphoreType.DMA,
        pltpu.SemaphoreType.DMA,
        pltpu.SemaphoreType.DMA,
        pltpu.SemaphoreType.DMA,
    ],
)
def _sc_aggregate(hw_hbm, gidx_hbm, dst_hbm, wedge_hbm, part_hbm,
                  agg_sh, wide2, gbuf2, dbuf2, dsc2, wsm2, gbt, dbt, wst,
                  gsem0, gsem1, isem0, isem1, ssem0, ssem1):
    c = lax.axis_index("c")
    s = lax.axis_index("s")
    wide = [wide2.at[0], wide2.at[1]]
    gbuf = [gbuf2.at[0], gbuf2.at[1]]
    dbuf = [dbuf2.at[0], dbuf2.at[1]]
    dsc = [dsc2.at[0], dsc2.at[1]]
    wsm = [wsm2.at[0], wsm2.at[1]]
    gsem = [gsem0, gsem1]
    isem = [isem0, isem1]
    ssem = [ssem0, ssem1]

    def zrow(i, _):
        for k in range(HP // 16):
            wide2[0, i, pl.ds(k * 16, 16)] = jnp.zeros((16,), jnp.float32)
        return 0
    lax.fori_loop(0, CHM, zrow, 0)
    row0 = pl.multiple_of(s * NT, 8)
    for off in range(0, NT, CHM):
        pltpu.sync_copy(wide[0], agg_sh.at[pl.ds(row0 + off, CHM)])
    plsc.subcore_barrier()

    wid = s * NC + c

    def ebase(j):
        return pl.multiple_of(wid * EW + j * CHM, 8)

    def issue_idx(j, sl):
        b = ebase(j)
        pltpu.async_copy(gidx_hbm.at[pl.ds(b, CHM)], gbuf[sl], isem[sl])
        pltpu.async_copy(dst_hbm.at[pl.ds(b, CHM)], dbuf[sl], isem[sl])
        pltpu.async_copy(wedge_hbm.at[pl.ds(b, CHM)], wsm[sl], isem[sl])

    def drain_idx(j, sl):
        b = ebase(j)
        pltpu.make_async_copy(gidx_hbm.at[pl.ds(b, CHM)], gbuf[sl], isem[sl]).wait()
        pltpu.make_async_copy(dst_hbm.at[pl.ds(b, CHM)], dbuf[sl], isem[sl]).wait()
        pltpu.make_async_copy(wedge_hbm.at[pl.ds(b, CHM)], wsm[sl], isem[sl]).wait()

    def scale_rows(wref, cn, wvals):
        def scale16(i, _):
            wv16 = wvals[pl.ds(i * 16, 16)]
            for u in range(16):
                e = i * 16 + u
                wv = wv16[u]
                for k in range(H // 16):
                    sl_ = pl.ds(k * 16, 16)
                    wref[e, sl_] = wref[e, sl_] * wv
            return 0
        lax.fori_loop(0, cn // 16, scale16, 0)

    b0 = ebase(0)
    pltpu.sync_copy(gidx_hbm.at[pl.ds(b0, CHM)], gbuf[0])
    pltpu.sync_copy(dst_hbm.at[pl.ds(b0, CHM)], dbuf[0])
    pltpu.sync_copy(wedge_hbm.at[pl.ds(b0, CHM)], wsm[0])
    pltpu.async_copy(hw_hbm.at[gbuf[0]], wide[0], gsem[0])
    issue_idx(1, 1)

    def pair(jj, _):
        for sl in (0, 1):
            j = jj * 2 + sl
            so = 1 - sl

            @pl.when(j + 1 < NFC)
            def _():
                drain_idx(j + 1, so)

            @pl.when(j >= 1)
            def _():
                pltpu.make_async_copy(
                    wide[so], agg_sh.at[dsc[so]], ssem[so]).wait()

            @pl.when(j + 1 < NFC)
            def _():
                pltpu.async_copy(hw_hbm.at[gbuf[so]], wide[so], gsem[so])

            pltpu.make_async_copy(hw_hbm.at[gbuf[sl]], wide[sl],
                                  gsem[sl]).wait()
            scale_rows(wide[sl], CHM, wsm[sl])

            def vcp(i, _):
                dsc2[sl, pl.ds(i * 16, 16)] = dbuf2[sl, pl.ds(i * 16, 16)]
                return 0
            lax.fori_loop(0, CHM // 16, vcp, 0)
            pltpu.async_copy(wide[sl], agg_sh.at[dsc[sl]], ssem[sl],
                             add=True)

            @pl.when(j + 2 < NFC)
            def _():
                issue_idx(j + 2, sl)
        return 0
    lax.fori_loop(0, NFC // 2, pair, 0)

    pltpu.make_async_copy(wide[1], agg_sh.at[dsc[1]], ssem[1]).wait()

    tbase = pl.multiple_of(wid * EW + NFC * CHM, 8)
    pltpu.sync_copy(gidx_hbm.at[pl.ds(tbase, TLM)], gbt)
    pltpu.sync_copy(dst_hbm.at[pl.ds(tbase, TLM)], dbt)
    pltpu.sync_copy(wedge_hbm.at[pl.ds(tbase, TLM)], wst)
    pltpu.async_copy(hw_hbm.at[gbt], wide[0].at[pl.ds(0, TLM)],
                     gsem[0]).wait()
    scale_rows(wide[0], TLM, wst)
    pltpu.sync_copy(wide[0].at[pl.ds(0, TLM)], agg_sh.at[dbt], add=True)

    plsc.subcore_barrier()
    pltpu.sync_copy(agg_sh.at[pl.ds(row0, NT)],
                    part_hbm.at[c, pl.ds(row0, NT)])


def _mlp_body(x_ref, w1_ref, b1_ref, w2_ref, b2_ref, o_ref):
    h = jnp.dot(x_ref[...], w1_ref[...], preferred_element_type=jnp.float32)
    h = jnp.maximum(h + b1_ref[...], 0.0)
    o_ref[...] = (jnp.dot(h, w2_ref[...], preferred_element_type=jnp.float32)
                  + b2_ref[...])


def _mlp_call(x, W1, b1, W2, b2):
    return pl.pallas_call(
        _mlp_body,
        out_shape=jax.ShapeDtypeStruct((N, H), jnp.float32),
    )(x, W1, b1, W2, b2)


def _einsum_body(h_ref, w_ref, o_ref):
    hw = jnp.dot(h_ref[...], w_ref[0], preferred_element_type=jnp.float32)
    o_ref[0] = jnp.concatenate(
        [hw, jnp.zeros((N, HP - H), jnp.float32)], axis=1)


def _einsum_call(h, Wrel_l):
    return pl.pallas_call(
        _einsum_body,
        grid=(R,),
        in_specs=[
            pl.BlockSpec((N, H), lambda r: (0, 0)),
            pl.BlockSpec((1, H, H), lambda r: (r, 0, 0)),
        ],
        out_specs=pl.BlockSpec((1, N, HP), lambda r: (r, 0, 0)),
        out_shape=jax.ShapeDtypeStruct((R, N, HP), jnp.float32),
    )(h, Wrel_l)


def _combine_body(part_ref, h_ref, wr_ref, b_ref, o_ref):
    o_ref[...] = (part_ref[0, :, :H] + part_ref[1, :, :H] + b_ref[...]
                  + jnp.dot(h_ref[...], wr_ref[...],
                            preferred_element_type=jnp.float32))


def _combine_call(part, h, Wroot_l, bconv_l):
    return pl.pallas_call(
        _combine_body,
        grid=(1,),
        in_specs=[
            pl.BlockSpec((NC, N, HP), lambda i: (0, 0, 0)),
            pl.BlockSpec((N, H), lambda i: (0, 0)),
            pl.BlockSpec((H, H), lambda i: (0, 0)),
            pl.BlockSpec((1, H), lambda i: (0, 0)),
        ],
        out_specs=pl.BlockSpec((N, H), lambda i: (0, 0)),
        out_shape=jax.ShapeDtypeStruct((N, H), jnp.float32),
    )(part, h, Wroot_l, bconv_l)


def _pool_body(h_ref, bat_ref, wo_ref, bo_ref, o_ref):
    h = h_ref[...]
    bat = bat_ref[...]
    cols = []
    for g in range(G):
        hm = jnp.where(bat == g, h, -jnp.inf)
        cols.append(jnp.max(hm, axis=0, keepdims=True))
    pooled = jnp.concatenate(cols, axis=0)
    o_ref[...] = (jnp.dot(pooled, wo_ref[...],
                          preferred_element_type=jnp.float32) + bo_ref[...])


def _pool_call(h, batch2d, Wout, bout):
    return pl.pallas_call(
        _pool_body,
        out_shape=jax.ShapeDtypeStruct((G, OUT), jnp.float32),
    )(h, batch2d, Wout, bout)


def kernel(x, edge_index, edge_type, batch,
           W1, b1, W2, b2, Wrel, Wroot, bconv, Wout, bout):
    src = edge_index[0]
    dst = edge_index[1]

    h = _mlp_call(x, W1, b1.reshape(1, H), W2, b2.reshape(1, H))
    _, wedge, gidx = _sc_precompute(src, dst, edge_type)

    for l in range(L):
        hw = _einsum_call(h, Wrel[l])
        part = _sc_aggregate(hw.reshape(R * N, HP), gidx, dst, wedge)
        h = _combine_call(part, h, Wroot[l], bconv[l].reshape(1, H))

    return _pool_call(h, batch.reshape(N, 1), Wout, bout.reshape(1, OUT))


# speedup vs baseline: 6.5777x; 1.0227x over previous
import functools

import jax
import jax.numpy as jnp
from jax import lax
from jax.experimental import pallas as pl
from jax.experimental.pallas import tpu as pltpu
from jax.experimental.pallas import tpu_sc as plsc

N = 10000
E = 320000
DIN = 128
H = 64
OUT = 64
R = 41
G = 8
L = 2

NC = 2
NS = 16
NW = NC * NS

HP = 128
NRP = 410112
WSL = NRP // NS
EC = E // NS
EW = E // NW
CHC = 2000
CHM = 128
NFC = EW // CHM
TLM = EW - NFC * CHM
NPAD = 10240
NT = NPAD // NS

_MESH = plsc.VectorSubcoreMesh(
    core_axis_name="c", subcore_axis_name="s", num_cores=NC, num_subcores=NS)


@functools.partial(
    pl.kernel,
    out_type=[
        jax.ShapeDtypeStruct((NC * NRP,), jnp.float32),
        jax.ShapeDtypeStruct((E,), jnp.float32),
        jax.ShapeDtypeStruct((E,), jnp.int32),
    ],
    mesh=_MESH,
    scratch_types=[
        pltpu.MemorySpace.VMEM_SHARED((NRP,), jnp.float32),
        pltpu.MemorySpace.VMEM((CHC,), jnp.int32),
        pltpu.MemorySpace.VMEM((CHC,), jnp.int32),
        pltpu.MemorySpace.VMEM((CHC,), jnp.int32),
        pltpu.MemorySpace.VMEM((CHC,), jnp.int32),
        pltpu.MemorySpace.VMEM((CHC,), jnp.int32),
        pltpu.MemorySpace.VMEM((CHC,), jnp.int32),
        pltpu.MemorySpace.VMEM((CHC,), jnp.float32),
        pltpu.MemorySpace.VMEM((WSL,), jnp.float32),
        pltpu.SemaphoreType.DMA,
        pltpu.SemaphoreType.DMA,
        pltpu.SemaphoreType.DMA,
    ],
)
def _sc_precompute(src_hbm, dst_hbm, typ_hbm, wtab_hbm, wedge_hbm, gidx_hbm,
                   cnt_sh, dbufA, dbufB, tbufA, tbufB, xbuf, ibuf, fbuf, cbuf,
                   sem, lsem0, lsem1):
    c = lax.axis_index("c")
    s = lax.axis_index("s")
    dbufs = [dbufA, dbufB]
    tbufs = [tbufA, tbufB]
    lsem = [lsem0, lsem1]
    dbuf = dbufA
    tbuf = tbufA

    wsl0 = pl.multiple_of(s * WSL, 8)

    def zero16(i, _):
        cbuf[pl.ds(i * 16, 16)] = jnp.zeros((16,), jnp.float32)
        return 0
    lax.fori_loop(0, WSL // 16, zero16, 0)
    pltpu.sync_copy(cbuf, cnt_sh.at[pl.ds(wsl0, WSL)])

    def ones16(i, _):
        fbuf[pl.ds(i * 16, 16)] = jnp.ones((16,), jnp.float32)
        return 0
    lax.fori_loop(0, CHC // 16, ones16, 0)
    plsc.subcore_barrier()

    NCC = EC // CHC

    def cbase(j):
        return pl.multiple_of(s * EC + j * CHC, 8)

    def cissue(j, sl):
        b = cbase(j)
        pltpu.async_copy(dst_hbm.at[pl.ds(b, CHC)], dbufs[sl], lsem[sl])
        pltpu.async_copy(typ_hbm.at[pl.ds(b, CHC)], tbufs[sl], lsem[sl])

    def cdrain(j, sl):
        b = cbase(j)
        pltpu.make_async_copy(dst_hbm.at[pl.ds(b, CHC)], dbufs[sl],
                              lsem[sl]).wait()
        pltpu.make_async_copy(typ_hbm.at[pl.ds(b, CHC)], tbufs[sl],
                              lsem[sl]).wait()

    cissue(0, 0)
    cissue(1, 1)

    def count_pair(jj, _):
        for sl in (0, 1):
            j = jj * 2 + sl
            cdrain(j, sl)

            def seg16(i, _):
                sl_ = pl.ds(i * 16, 16)
                ibuf[sl_] = dbufs[sl][sl_] * R + tbufs[sl][sl_]
                return 0
            lax.fori_loop(0, CHC // 16, seg16, 0)
            pltpu.sync_copy(fbuf, cnt_sh.at[ibuf], add=True)

            @pl.when(j + 2 < NCC)
            def _():
                cissue(j + 2, sl)
        return 0
    lax.fori_loop(0, NCC // 2, count_pair, 0)
    plsc.subcore_barrier()

    pltpu.sync_copy(cnt_sh.at[pl.ds(wsl0, WSL)], cbuf)

    def recip16(i, _):
        sl = pl.ds(i * 16, 16)
        cbuf[sl] = 1.0 / jnp.maximum(cbuf[sl], 1.0)
        return 0
    lax.fori_loop(0, WSL // 16, recip16, 0)
    wbase = pl.multiple_of(c * NRP + s * WSL, 8)
    pltpu.sync_copy(cbuf, wtab_hbm.at[pl.ds(wbase, WSL)])
    plsc.subcore_barrier()

    wid = s * NC + c

    def edge_chunk(j, _):
        base = pl.multiple_of(wid * EW + j * CHC, 8)
        pltpu.sync_copy(src_hbm.at[pl.ds(base, CHC)], xbuf)
        pltpu.sync_copy(dst_hbm.at[pl.ds(base, CHC)], dbuf)
        pltpu.sync_copy(typ_hbm.at[pl.ds(base, CHC)], tbuf)

        def gi16(i, _):
            sl = pl.ds(i * 16, 16)
            t = tbuf[sl]
            ibuf[sl] = dbuf[sl] * R + t + c * NRP
            xbuf[sl] = t * N + xbuf[sl]
            return 0
        lax.fori_loop(0, CHC // 16, gi16, 0)
        pltpu.sync_copy(xbuf, gidx_hbm.at[pl.ds(base, CHC)])
        pltpu.async_copy(wtab_hbm.at[ibuf], fbuf, sem).wait()
        pltpu.sync_copy(fbuf, wedge_hbm.at[pl.ds(base, CHC)])
        return 0
    lax.fori_loop(0, EW // CHC, edge_chunk, 0)


@functools.partial(
    pl.kernel,
    out_type=jax.ShapeDtypeStruct((NC, NPAD, HP), jnp.float32),
    mesh=_MESH,
    scratch_types=[
        pltpu.MemorySpace.VMEM_SHARED((NPAD, HP), jnp.float32),
        pltpu.MemorySpace.VMEM((2, CHM, HP), jnp.float32),
        pltpu.MemorySpace.VMEM((2, CHM), jnp.int32),
        pltpu.MemorySpace.VMEM((2, CHM), jnp.int32),
        pltpu.MemorySpace.VMEM((2, CHM), jnp.int32),
        pltpu.MemorySpace.VMEM((2, CHM), jnp.float32),
        pltpu.MemorySpace.VMEM((TLM,), jnp.int32),
        pltpu.MemorySpace.VMEM((TLM,), jnp.int32),
        pltpu.MemorySpace.VMEM((TLM,), jnp.float32),
        pltpu.SemaphoreType.DMA,
        pltpu.SemaphoreType.DMA,
        pltpu.SemaphoreType.DMA,
        pltpu.SemaphoreType.DMA,
        pltpu.SemaphoreType.DMA,
        pltpu.SemaphoreType.DMA,
    ],
)
def _sc_aggregate(hw_hbm, gidx_hbm, dst_hbm, wedge_hbm, part_hbm,
                  agg_sh, wide2, gbuf2, dbuf2, dsc2, wsm2, gbt, dbt, wst,
                  gsem0, gsem1, isem0, isem1, ssem0, ssem1):
    c = lax.axis_index("c")
    s = lax.axis_index("s")
    wide = [wide2.at[0], wide2.at[1]]
    gbuf = [gbuf2.at[0], gbuf2.at[1]]
    dbuf = [dbuf2.at[0], dbuf2.at[1]]
    dsc = [dsc2.at[0], dsc2.at[1]]
    wsm = [wsm2.at[0], wsm2.at[1]]
    gsem = [gsem0, gsem1]
    isem = [isem0, isem1]
    ssem = [ssem0, ssem1]

    def zrow(i, _):
        for k in range(HP // 16):
            wide2[0, i, pl.ds(k * 16, 16)] = jnp.zeros((16,), jnp.float32)
        return 0
    lax.fori_loop(0, CHM, zrow, 0)
    row0 = pl.multiple_of(s * NT, 8)
    for off in range(0, NT, CHM):
        pltpu.sync_copy(wide[0], agg_sh.at[pl.ds(row0 + off, CHM)])
    plsc.subcore_barrier()

    wid = s * NC + c

    def ebase(j):
        return pl.multiple_of(wid * EW + j * CHM, 8)

    def issue_idx(j, sl):
        b = ebase(j)
        pltpu.async_copy(gidx_hbm.at[pl.ds(b, CHM)], gbuf[sl], isem[sl])
        pltpu.async_copy(dst_hbm.at[pl.ds(b, CHM)], dbuf[sl], isem[sl])
        pltpu.async_copy(wedge_hbm.at[pl.ds(b, CHM)], wsm[sl], isem[sl])

    def drain_idx(j, sl):
        b = ebase(j)
        pltpu.make_async_copy(gidx_hbm.at[pl.ds(b, CHM)], gbuf[sl], isem[sl]).wait()
        pltpu.make_async_copy(dst_hbm.at[pl.ds(b, CHM)], dbuf[sl], isem[sl]).wait()
        pltpu.make_async_copy(wedge_hbm.at[pl.ds(b, CHM)], wsm[sl], isem[sl]).wait()

    def scale_rows(wref, cn, wvals):
        def scale16(i, _):
            wv16 = wvals[pl.ds(i * 16, 16)]
            for u in range(16):
                e = i * 16 + u
                wv = wv16[u]
                for k in range(H // 16):
                    sl_ = pl.ds(k * 16, 16)
                    wref[e, sl_] = wref[e, sl_] * wv
            return 0
        lax.fori_loop(0, cn // 16, scale16, 0)

    b0 = ebase(0)
    pltpu.sync_copy(gidx_hbm.at[pl.ds(b0, CHM)], gbuf[0])
    pltpu.sync_copy(dst_hbm.at[pl.ds(b0, CHM)], dbuf[0])
    pltpu.sync_copy(wedge_hbm.at[pl.ds(b0, CHM)], wsm[0])
    pltpu.async_copy(hw_hbm.at[gbuf[0]], wide[0], gsem[0])
    issue_idx(1, 1)

    def pair(jj, _):
        for sl in (0, 1):
            j = jj * 2 + sl
            so = 1 - sl

            @pl.when(j + 1 < NFC)
            def _():
                drain_idx(j + 1, so)

            @pl.when(j >= 1)
            def _():
                pltpu.make_async_copy(
                    wide[so], agg_sh.at[dsc[so]], ssem[so]).wait()

            @pl.when(j + 1 < NFC)
            def _():
                pltpu.async_copy(hw_hbm.at[gbuf[so]], wide[so], gsem[so])

            pltpu.make_async_copy(hw_hbm.at[gbuf[sl]], wide[sl],
                                  gsem[sl]).wait()
            scale_rows(wide[sl], CHM, wsm[sl])

            def vcp(i, _):
                dsc2[sl, pl.ds(i * 16, 16)] = dbuf2[sl, pl.ds(i * 16, 16)]
                return 0
            lax.fori_loop(0, CHM // 16, vcp, 0)
            pltpu.async_copy(wide[sl], agg_sh.at[dsc[sl]], ssem[sl],
                             add=True)

            @pl.when(j + 2 < NFC)
            def _():
                issue_idx(j + 2, sl)
        return 0
    lax.fori_loop(0, NFC // 2, pair, 0)

    pltpu.make_async_copy(wide[1], agg_sh.at[dsc[1]], ssem[1]).wait()

    tbase = pl.multiple_of(wid * EW + NFC * CHM, 8)
    pltpu.sync_copy(gidx_hbm.at[pl.ds(tbase, TLM)], gbt)
    pltpu.sync_copy(dst_hbm.at[pl.ds(tbase, TLM)], dbt)
    pltpu.sync_copy(wedge_hbm.at[pl.ds(tbase, TLM)], wst)
    pltpu.async_copy(hw_hbm.at[gbt], wide[0].at[pl.ds(0, TLM)],
                     gsem[0]).wait()
    scale_rows(wide[0], TLM, wst)
    pltpu.sync_copy(wide[0].at[pl.ds(0, TLM)], agg_sh.at[dbt], add=True)

    plsc.subcore_barrier()
    pltpu.sync_copy(agg_sh.at[pl.ds(row0, NT)],
                    part_hbm.at[c, pl.ds(row0, NT)])


def _mlp_body(x_ref, w1_ref, b1_ref, w2_ref, b2_ref, o_ref):
    h = jnp.dot(x_ref[...], w1_ref[...], preferred_element_type=jnp.float32)
    h = jnp.maximum(h + b1_ref[...], 0.0)
    o_ref[...] = (jnp.dot(h, w2_ref[...], preferred_element_type=jnp.float32)
                  + b2_ref[...])


def _mlp_call(x, W1, b1, W2, b2):
    return pl.pallas_call(
        _mlp_body,
        out_shape=jax.ShapeDtypeStruct((N, H), jnp.float32),
    )(x, W1, b1, W2, b2)


def _einsum_body(h_ref, w_ref, o_ref):
    hw = jnp.dot(h_ref[...], w_ref[0], preferred_element_type=jnp.float32)
    o_ref[0] = jnp.concatenate(
        [hw, jnp.zeros((N, HP - H), jnp.float32)], axis=1)


def _einsum_call(h, Wrel_l):
    return pl.pallas_call(
        _einsum_body,
        grid=(R,),
        in_specs=[
            pl.BlockSpec((N, H), lambda r: (0, 0)),
            pl.BlockSpec((1, H, H), lambda r: (r, 0, 0)),
        ],
        out_specs=pl.BlockSpec((1, N, HP), lambda r: (r, 0, 0)),
        out_shape=jax.ShapeDtypeStruct((R, N, HP), jnp.float32),
    )(h, Wrel_l)


def _combine_body(part_ref, h_ref, wr_ref, b_ref, o_ref):
    o_ref[...] = (part_ref[0, :, :H] + part_ref[1, :, :H] + b_ref[...]
                  + jnp.dot(h_ref[...], wr_ref[...],
                            preferred_element_type=jnp.float32))


def _combine_call(part, h, Wroot_l, bconv_l):
    return pl.pallas_call(
        _combine_body,
        grid=(1,),
        in_specs=[
            pl.BlockSpec((NC, N, HP), lambda i: (0, 0, 0)),
            pl.BlockSpec((N, H), lambda i: (0, 0)),
            pl.BlockSpec((H, H), lambda i: (0, 0)),
            pl.BlockSpec((1, H), lambda i: (0, 0)),
        ],
        out_specs=pl.BlockSpec((N, H), lambda i: (0, 0)),
        out_shape=jax.ShapeDtypeStruct((N, H), jnp.float32),
    )(part, h, Wroot_l, bconv_l)


def _pool_body(h_ref, bat_ref, wo_ref, bo_ref, o_ref):
    h = h_ref[...]
    bat = bat_ref[...]
    cols = []
    for g in range(G):
        hm = jnp.where(bat == g, h, -jnp.inf)
        cols.append(jnp.max(hm, axis=0, keepdims=True))
    pooled = jnp.concatenate(cols, axis=0)
    o_ref[...] = (jnp.dot(pooled, wo_ref[...],
                          preferred_element_type=jnp.float32) + bo_ref[...])


def _pool_call(h, batch2d, Wout, bout):
    return pl.pallas_call(
        _pool_body,
        out_shape=jax.ShapeDtypeStruct((G, OUT), jnp.float32),
    )(h, batch2d, Wout, bout)


def kernel(x, edge_index, edge_type, batch,
           W1, b1, W2, b2, Wrel, Wroot, bconv, Wout, bout):
    src = edge_index[0]
    dst = edge_index[1]

    h = _mlp_call(x, W1, b1.reshape(1, H), W2, b2.reshape(1, H))
    _, wedge, gidx = _sc_precompute(src, dst, edge_type)

    for l in range(L):
        hw = _einsum_call(h, Wrel[l])
        part = _sc_aggregate(hw.reshape(R * N, HP), gidx, dst, wedge)
        h = _combine_call(part, h, Wroot[l], bconv[l].reshape(1, H))

    return _pool_call(h, batch.reshape(N, 1), Wout, bout.reshape(1, OUT))
